# baseline (speedup 1.0000x reference)
import functools

import jax
import jax.numpy as jnp
from jax import lax
from jax.experimental import pallas as pl
from jax.experimental.pallas import tpu as pltpu
from jax.experimental.pallas import tpu_sc as plsc

_RR = 0.25
_K = 32
_N = 2048
_M = 256
_TP = 8
_DIM = 128
_NW = 32
_GROUPS = 96


def _fps_body(p_ref, ax_ref, ay_ref, az_ref):
    px = p_ref[0]
    py = p_ref[1]
    pz = p_ref[2]
    iota_n = lax.broadcasted_iota(jnp.int32, (32, _N), 1)
    iota_m = lax.broadcasted_iota(jnp.int32, (32, _M), 1)

    lx0 = px[:, 0:1]
    ly0 = py[:, 0:1]
    lz0 = pz[:, 0:1]
    ax = jnp.where(iota_m == 0, lx0, 0.0)
    ay = jnp.where(iota_m == 0, ly0, 0.0)
    az = jnp.where(iota_m == 0, lz0, 0.0)
    dists = jnp.full((32, _N), 1e10, dtype=jnp.float32)

    def body(i, st):
        dists, lx, ly, lz, ax, ay, az = st
        dx = px - lx
        dy = py - ly
        dz = pz - lz
        d = dx * dx + dy * dy + dz * dz
        dists = jnp.minimum(dists, d)
        mx = jnp.max(dists, axis=1, keepdims=True)
        nxt = jnp.min(jnp.where(dists == mx, iota_n, _N), axis=1, keepdims=True)
        oh = iota_n == nxt
        nlx = jnp.sum(jnp.where(oh, px, 0.0), axis=1, keepdims=True)
        nly = jnp.sum(jnp.where(oh, py, 0.0), axis=1, keepdims=True)
        nlz = jnp.sum(jnp.where(oh, pz, 0.0), axis=1, keepdims=True)
        ohc = iota_m == i
        ax = jnp.where(ohc, nlx, ax)
        ay = jnp.where(ohc, nly, ay)
        az = jnp.where(ohc, nlz, az)
        return dists, nlx, nly, nlz, ax, ay, az

    st = (dists, lx0, ly0, lz0, ax, ay, az)
    st = lax.fori_loop(1, _M, body, st)
    ax_ref[...] = st[4]
    ay_ref[...] = st[5]
    az_ref[...] = st[6]


def _run_fps(p_soa):
    shp = jax.ShapeDtypeStruct((32, _M), jnp.float32)
    return pl.pallas_call(
        _fps_body,
        out_shape=(shp, shp, shp),
    )(p_soa)


def _g_body(x_ref, v_ref, o_ref):
    o_ref[...] = jnp.dot(x_ref[...], v_ref[...],
                         preferred_element_type=jnp.float32)


def _run_g(x_flat, vt):
    rows = x_flat.shape[0]
    bs = 8192
    return pl.pallas_call(
        _g_body,
        grid=(rows // bs,),
        in_specs=[
            pl.BlockSpec((bs, 3), lambda i: (i, 0)),
            pl.BlockSpec((3, _DIM), lambda i: (0, 0)),
        ],
        out_specs=pl.BlockSpec((bs, _DIM), lambda i: (i, 0)),
        out_shape=jax.ShapeDtypeStruct((rows, _DIM), jnp.float32),
    )(x_flat, vt)


def _sc_kernel_fn():
    mesh = plsc.VectorSubcoreMesh(core_axis_name="c", subcore_axis_name="s")

    @functools.partial(
        pl.kernel,
        mesh=mesh,
        out_type=jax.ShapeDtypeStruct((_GROUPS * _M * _DIM,), jnp.float32),
        scratch_types=[
            pltpu.VMEM((3 * _N,), jnp.float32),
            pltpu.VMEM((3 * _M + 32,), jnp.float32),
            pltpu.VMEM((48,), jnp.int32),
            pltpu.VMEM((64,), jnp.int32),
            pltpu.VMEM((_K,), jnp.int32),
            pltpu.VMEM((_K, _DIM), jnp.float32),
            pltpu.VMEM((_M * _DIM,), jnp.float32),
            pltpu.SemaphoreType.DMA,
        ],
    )
    def sc_kern(pts_hbm, anc_hbm, g_hbm, out_hbm,
                pts_v, anc_v, wrk_v, idx_v, gid_v, rows_v, out_v, sem):
        wid = lax.axis_index("s") * 2 + lax.axis_index("c")
        lane = lax.broadcasted_iota(jnp.int32, (16,), 0)

        for k in range(_GROUPS // _NW):
            gid = wid * (_GROUPS // _NW) + k
            b = gid // 24
            rem = gid - b * 24
            o = rem // 3
            j = rem - o * 3
            frame = jnp.clip(2 * o - 1 + j, 0, 15)
            base = (b * 16 + frame) * _N

            for cc in range(3):
                pltpu.sync_copy(
                    pts_hbm.at[pl.ds(((cc * 4 + b) * 16 + frame) * _N, _N)],
                    pts_v.at[pl.ds(cc * _N, _N)])
                pltpu.sync_copy(
                    anc_hbm.at[pl.ds(((b * 8 + o) * 3 + cc) * _M, _M)],
                    anc_v.at[pl.ds(cc * _M, _M)])

            def per_anchor(a, carry):
                axv = jnp.full((16,), anc_v[pl.ds(a, 16)][0], jnp.float32)
                ayv = jnp.full((16,), anc_v[pl.ds(_M + a, 16)][0],
                               jnp.float32)
                azv = jnp.full((16,), anc_v[pl.ds(2 * _M + a, 16)][0],
                               jnp.float32)

                def scan_body(ch, cnt):
                    def hot():
                        px = pts_v[pl.ds(ch * 16, 16)]
                        py = pts_v[pl.ds(_N + ch * 16, 16)]
                        pz = pts_v[pl.ds(2 * _N + ch * 16, 16)]
                        dx = px - axv
                        dy = py - ayv
                        dz = pz - azv
                        d2 = dx * dx + dy * dy + dz * dz
                        m = d2 < _RR
                        mi = jnp.where(m, 1, 0)
                        s = mi
                        for sh in (1, 2, 4, 8):
                            wrk_v[pl.ds(16, 16)] = s
                            s = s + jnp.where(lane >= sh,
                                              wrk_v[pl.ds(16 - sh, 16)], 0)
                        tot = s[15]

                        def app(c):
                            osel = jnp.where(m, c + s - 1, 48)
                            base_i = ch * 16
                            for l in range(16):
                                idx_v[pl.ds(osel[l], 16)] = jnp.full(
                                    (16,), base_i + l, jnp.int32)
                            return c + tot

                        return lax.cond(tot > 0, app, lambda c: c, cnt)

                    return lax.cond(cnt < _K, hot, lambda: cnt)

                cnt = lax.fori_loop(0, _N // 16, scan_body, jnp.int32(0))

                v0 = idx_v[pl.ds(0, 16)]
                v1 = idx_v[pl.ds(16, 16)]
                i0 = jnp.where(cnt > 0, v0[0], 0)
                s0 = jnp.where(lane < cnt, v0, i0) + base
                s1 = jnp.where(lane + 16 < cnt, v1, i0) + base
                gid_v[pl.ds(0, 16)] = s0
                gid_v[pl.ds(16, 16)] = s1

                pltpu.async_copy(g_hbm.at[gid_v], rows_v, sem).wait()

                acc = [rows_v[0, pl.ds(c * 16, 16)] for c in range(8)]
                for r in range(1, _K):
                    for c in range(8):
                        acc[c] = jnp.maximum(acc[c],
                                             rows_v[r, pl.ds(c * 16, 16)])
                for c in range(8):
                    out_v[pl.ds(a * _DIM + c * 16, 16)] = acc[c]
                return carry

            lax.fori_loop(0, _M, per_anchor, jnp.int32(0))
            pltpu.sync_copy(out_v, out_hbm.at[pl.ds(gid * _M * _DIM,
                                                    _M * _DIM)])

    return sc_kern


def _epi_body(s0_ref, s1_ref, s2_ref, a2_ref, wdt_ref, wpt_ref, bp_ref,
              o_ref):
    g = pl.program_id(0)
    ts = (g % _TP + 1).astype(jnp.float32)
    wd3 = wdt_ref[3:4, :]
    m = jnp.maximum(jnp.maximum(s0_ref[...] - wd3, s1_ref[...]),
                    s2_ref[...] + wd3)
    a2 = a2_ref[...]
    neg = jnp.dot(a2, wdt_ref[0:3, :], preferred_element_type=jnp.float32)
    pos = jnp.dot(a2, wpt_ref[0:3, :], preferred_element_type=jnp.float32)
    emb = m - neg + pos + ts * wpt_ref[3:4, :] + bp_ref[...]
    o_ref[...] = jnp.maximum(emb, 0.0)


def _run_epilogue(scmax_rows, a2, wdt, wpt, bp):
    def im(j):
        return lambda g: ((g // _TP) * 24 + (g % _TP) * 3 + j, 0)

    return pl.pallas_call(
        _epi_body,
        grid=(32,),
        in_specs=[
            pl.BlockSpec((_M, _DIM), im(0)),
            pl.BlockSpec((_M, _DIM), im(1)),
            pl.BlockSpec((_M, _DIM), im(2)),
            pl.BlockSpec((_M, 3), lambda g: (g, 0)),
            pl.BlockSpec((4, _DIM), lambda g: (0, 0)),
            pl.BlockSpec((4, _DIM), lambda g: (0, 0)),
            pl.BlockSpec((1, _DIM), lambda g: (0, 0)),
        ],
        out_specs=pl.BlockSpec((_M, _DIM), lambda g: (g, 0)),
        out_shape=jax.ShapeDtypeStruct((32 * _M, _DIM), jnp.float32),
    )(scmax_rows, scmax_rows, scmax_rows, a2, wdt, wpt, bp)


@jax.jit
def kernel(x, W_d, W_f, W_pos, b_pos):
    B, T, N, _ = x.shape

    pts_soa = jnp.transpose(x, (3, 0, 1, 2))
    p_fps = pts_soa[:, :, ::2].reshape(3, 32, N)
    x_flat = x.reshape(B * T * N, 3)
    vt = jnp.concatenate([W_d[:, :2], W_d[:, 2:3] + W_f], axis=1).T
    wdt = W_d.T
    wpt = W_pos.T
    bp = b_pos[None, :]

    ax, ay, az = _run_fps(p_fps)
    anc = jnp.stack([ax, ay, az], axis=1).reshape(B, _TP, 3, _M)
    a2 = jnp.stack([ax, ay, az], axis=2).reshape(32 * _M, 3)

    g = _run_g(x_flat, vt)

    pts_1d = pts_soa.reshape(-1)
    anc_1d = anc.reshape(-1)
    scmax = _sc_kernel_fn()(pts_1d, anc_1d, g)
    scmax_rows = scmax.reshape(_GROUPS * _M, _DIM)

    emb = _run_epilogue(scmax_rows, a2, wdt, wpt, bp)
    return emb.reshape(B, _TP * _M, _DIM)


# speedup vs baseline: 1.1138x; 1.1138x over previous
import functools

import jax
import jax.numpy as jnp
from jax import lax
from jax.experimental import pallas as pl
from jax.experimental.pallas import tpu as pltpu
from jax.experimental.pallas import tpu_sc as plsc

_RR = 0.25
_K = 32
_N = 2048
_M = 256
_TP = 8
_DIM = 128
_NW = 32
_GROUPS = 96


def _fps_body(p_ref, ax_ref, ay_ref, az_ref):
    px = p_ref[0]
    py = p_ref[1]
    pz = p_ref[2]
    iota_n = lax.broadcasted_iota(jnp.int32, (32, _N), 1)
    iota_m = lax.broadcasted_iota(jnp.int32, (32, _M), 1)

    lx0 = px[:, 0:1]
    ly0 = py[:, 0:1]
    lz0 = pz[:, 0:1]
    ax = jnp.where(iota_m == 0, lx0, 0.0)
    ay = jnp.where(iota_m == 0, ly0, 0.0)
    az = jnp.where(iota_m == 0, lz0, 0.0)
    dists = jnp.full((32, _N), 1e10, dtype=jnp.float32)

    def body(i, st):
        dists, lx, ly, lz, ax, ay, az = st
        dx = px - lx
        dy = py - ly
        dz = pz - lz
        d = dx * dx + dy * dy + dz * dz
        dists = jnp.minimum(dists, d)
        mx = jnp.max(dists, axis=1, keepdims=True)
        nxt = jnp.min(jnp.where(dists == mx, iota_n, _N), axis=1, keepdims=True)
        oh = iota_n == nxt
        nlx = jnp.sum(jnp.where(oh, px, 0.0), axis=1, keepdims=True)
        nly = jnp.sum(jnp.where(oh, py, 0.0), axis=1, keepdims=True)
        nlz = jnp.sum(jnp.where(oh, pz, 0.0), axis=1, keepdims=True)
        ohc = iota_m == i
        ax = jnp.where(ohc, nlx, ax)
        ay = jnp.where(ohc, nly, ay)
        az = jnp.where(ohc, nlz, az)
        return dists, nlx, nly, nlz, ax, ay, az

    st = (dists, lx0, ly0, lz0, ax, ay, az)
    st = lax.fori_loop(1, _M, body, st)
    ax_ref[...] = st[4]
    ay_ref[...] = st[5]
    az_ref[...] = st[6]


def _run_fps(p_soa):
    shp = jax.ShapeDtypeStruct((32, _M), jnp.float32)
    return pl.pallas_call(
        _fps_body,
        out_shape=(shp, shp, shp),
    )(p_soa)


def _g_body(x_ref, v_ref, o_ref):
    o_ref[...] = jnp.dot(x_ref[...], v_ref[...],
                         preferred_element_type=jnp.float32)


def _run_g(x_flat, vt):
    rows = x_flat.shape[0]
    bs = 8192
    return pl.pallas_call(
        _g_body,
        grid=(rows // bs,),
        in_specs=[
            pl.BlockSpec((bs, 3), lambda i: (i, 0)),
            pl.BlockSpec((3, _DIM), lambda i: (0, 0)),
        ],
        out_specs=pl.BlockSpec((bs, _DIM), lambda i: (i, 0)),
        out_shape=jax.ShapeDtypeStruct((rows, _DIM), jnp.float32),
    )(x_flat, vt)


def _sc_kernel_fn():
    mesh = plsc.VectorSubcoreMesh(core_axis_name="c", subcore_axis_name="s")

    @functools.partial(
        pl.kernel,
        mesh=mesh,
        out_type=jax.ShapeDtypeStruct((_GROUPS * _M * _DIM,), jnp.float32),
        scratch_types=[
            pltpu.VMEM((3 * _N,), jnp.float32),
            pltpu.VMEM((3 * _M + 32,), jnp.float32),
            pltpu.VMEM((48,), jnp.int32),
            pltpu.VMEM((64,), jnp.int32),
            pltpu.VMEM((_K,), jnp.int32),
            pltpu.VMEM((_K, _DIM), jnp.float32),
            pltpu.VMEM((_M * _DIM,), jnp.float32),
            pltpu.SemaphoreType.DMA,
        ],
    )
    def sc_kern(pts_hbm, anc_hbm, g_hbm, out_hbm,
                pts_v, anc_v, wrk_v, idx_v, gid_v, rows_v, out_v, sem):
        wid = lax.axis_index("s") * 2 + lax.axis_index("c")
        lane = lax.broadcasted_iota(jnp.int32, (16,), 0)

        for k in range(_GROUPS // _NW):
            gid = wid * (_GROUPS // _NW) + k
            b = gid // 24
            rem = gid - b * 24
            o = rem // 3
            j = rem - o * 3
            frame = jnp.clip(2 * o - 1 + j, 0, 15)
            base = (b * 16 + frame) * _N

            for cc in range(3):
                pltpu.sync_copy(
                    pts_hbm.at[pl.ds(((cc * 4 + b) * 16 + frame) * _N, _N)],
                    pts_v.at[pl.ds(cc * _N, _N)])
                pltpu.sync_copy(
                    anc_hbm.at[pl.ds(((b * 8 + o) * 3 + cc) * _M, _M)],
                    anc_v.at[pl.ds(cc * _M, _M)])

            def per_anchor(a, carry):
                axv = jnp.full((16,), anc_v[pl.ds(a, 16)][0], jnp.float32)
                ayv = jnp.full((16,), anc_v[pl.ds(_M + a, 16)][0],
                               jnp.float32)
                azv = jnp.full((16,), anc_v[pl.ds(2 * _M + a, 16)][0],
                               jnp.float32)

                def scan_body(ch, cnt):
                    def hot():
                        px = pts_v[pl.ds(ch * 16, 16)]
                        py = pts_v[pl.ds(_N + ch * 16, 16)]
                        pz = pts_v[pl.ds(2 * _N + ch * 16, 16)]
                        dx = px - axv
                        dy = py - ayv
                        dz = pz - azv
                        d2 = dx * dx + dy * dy + dz * dz
                        m = d2 < _RR
                        mi = jnp.where(m, 1, 0)
                        wrk_v[pl.ds(16, 16)] = mi
                        s = mi + jnp.where(lane < 12,
                                           wrk_v[pl.ds(20, 16)], 0)
                        wrk_v[pl.ds(16, 16)] = s
                        s = s + jnp.where(lane < 8,
                                          wrk_v[pl.ds(24, 16)], 0)
                        tot = s[0] + s[1] + s[2] + s[3]

                        def app(c):
                            for l in range(16):
                                @pl.when(mi[l] == 1)
                                def _(l=l, c=c):
                                    idx_v[pl.ds(c, 16)] = jnp.full(
                                        (16,), ch * 16 + l, jnp.int32)
                                c = c + mi[l]
                            return c

                        return lax.cond(tot > 0, app, lambda c: c, cnt)

                    return lax.cond(cnt < _K, hot, lambda: cnt)

                cnt = lax.fori_loop(0, _N // 16, scan_body, jnp.int32(0))

                v0 = idx_v[pl.ds(0, 16)]
                v1 = idx_v[pl.ds(16, 16)]
                i0 = jnp.where(cnt > 0, v0[0], 0)
                s0 = jnp.where(lane < cnt, v0, i0) + base
                s1 = jnp.where(lane + 16 < cnt, v1, i0) + base
                gid_v[pl.ds(0, 16)] = s0
                gid_v[pl.ds(16, 16)] = s1

                pltpu.async_copy(g_hbm.at[gid_v], rows_v, sem).wait()

                acc = [rows_v[0, pl.ds(c * 16, 16)] for c in range(8)]
                for r in range(1, _K):
                    for c in range(8):
                        acc[c] = jnp.maximum(acc[c],
                                             rows_v[r, pl.ds(c * 16, 16)])
                for c in range(8):
                    out_v[pl.ds(a * _DIM + c * 16, 16)] = acc[c]
                return carry

            lax.fori_loop(0, _M, per_anchor, jnp.int32(0))
            pltpu.sync_copy(out_v, out_hbm.at[pl.ds(gid * _M * _DIM,
                                                    _M * _DIM)])

    return sc_kern


def _epi_body(s0_ref, s1_ref, s2_ref, a2_ref, wdt_ref, wpt_ref, bp_ref,
              o_ref):
    g = pl.program_id(0)
    ts = (g % _TP + 1).astype(jnp.float32)
    wd3 = wdt_ref[3:4, :]
    m = jnp.maximum(jnp.maximum(s0_ref[...] - wd3, s1_ref[...]),
                    s2_ref[...] + wd3)
    a2 = a2_ref[...]
    neg = jnp.dot(a2, wdt_ref[0:3, :], preferred_element_type=jnp.float32)
    pos = jnp.dot(a2, wpt_ref[0:3, :], preferred_element_type=jnp.float32)
    emb = m - neg + pos + ts * wpt_ref[3:4, :] + bp_ref[...]
    o_ref[...] = jnp.maximum(emb, 0.0)


def _run_epilogue(scmax_rows, a2, wdt, wpt, bp):
    def im(j):
        return lambda g: ((g // _TP) * 24 + (g % _TP) * 3 + j, 0)

    return pl.pallas_call(
        _epi_body,
        grid=(32,),
        in_specs=[
            pl.BlockSpec((_M, _DIM), im(0)),
            pl.BlockSpec((_M, _DIM), im(1)),
            pl.BlockSpec((_M, _DIM), im(2)),
            pl.BlockSpec((_M, 3), lambda g: (g, 0)),
            pl.BlockSpec((4, _DIM), lambda g: (0, 0)),
            pl.BlockSpec((4, _DIM), lambda g: (0, 0)),
            pl.BlockSpec((1, _DIM), lambda g: (0, 0)),
        ],
        out_specs=pl.BlockSpec((_M, _DIM), lambda g: (g, 0)),
        out_shape=jax.ShapeDtypeStruct((32 * _M, _DIM), jnp.float32),
    )(scmax_rows, scmax_rows, scmax_rows, a2, wdt, wpt, bp)


@jax.jit
def kernel(x, W_d, W_f, W_pos, b_pos):
    B, T, N, _ = x.shape

    pts_soa = jnp.transpose(x, (3, 0, 1, 2))
    p_fps = pts_soa[:, :, ::2].reshape(3, 32, N)
    x_flat = x.reshape(B * T * N, 3)
    vt = jnp.concatenate([W_d[:, :2], W_d[:, 2:3] + W_f], axis=1).T
    wdt = W_d.T
    wpt = W_pos.T
    bp = b_pos[None, :]

    ax, ay, az = _run_fps(p_fps)
    anc = jnp.stack([ax, ay, az], axis=1).reshape(B, _TP, 3, _M)
    a2 = jnp.stack([ax, ay, az], axis=2).reshape(32 * _M, 3)

    g = _run_g(x_flat, vt)

    pts_1d = pts_soa.reshape(-1)
    anc_1d = anc.reshape(-1)
    scmax = _sc_kernel_fn()(pts_1d, anc_1d, g)
    scmax_rows = scmax.reshape(_GROUPS * _M, _DIM)

    emb = _run_epilogue(scmax_rows, a2, wdt, wpt, bp)
    return emb.reshape(B, _TP * _M, _DIM)


# speedup vs baseline: 1.2225x; 1.0976x over previous
import functools

import jax
import jax.numpy as jnp
from jax import lax
from jax.experimental import pallas as pl
from jax.experimental.pallas import tpu as pltpu
from jax.experimental.pallas import tpu_sc as plsc

_RR = 0.25
_K = 32
_N = 2048
_M = 256
_TP = 8
_DIM = 128
_NW = 32
_GROUPS = 96


def _fps_body(p_ref, ax_ref, ay_ref, az_ref):
    px = p_ref[0]
    py = p_ref[1]
    pz = p_ref[2]
    iota_n = lax.broadcasted_iota(jnp.int32, (32, _N), 1)
    iota_m = lax.broadcasted_iota(jnp.int32, (32, _M), 1)

    lx0 = px[:, 0:1]
    ly0 = py[:, 0:1]
    lz0 = pz[:, 0:1]
    ax = jnp.where(iota_m == 0, lx0, 0.0)
    ay = jnp.where(iota_m == 0, ly0, 0.0)
    az = jnp.where(iota_m == 0, lz0, 0.0)
    dists = jnp.full((32, _N), 1e10, dtype=jnp.float32)

    def body(i, st):
        dists, lx, ly, lz, ax, ay, az = st
        dx = px - lx
        dy = py - ly
        dz = pz - lz
        d = dx * dx + dy * dy + dz * dz
        dists = jnp.minimum(dists, d)
        mx = jnp.max(dists, axis=1, keepdims=True)
        nxt = jnp.min(jnp.where(dists == mx, iota_n, _N), axis=1, keepdims=True)
        oh = iota_n == nxt
        nlx = jnp.sum(jnp.where(oh, px, 0.0), axis=1, keepdims=True)
        nly = jnp.sum(jnp.where(oh, py, 0.0), axis=1, keepdims=True)
        nlz = jnp.sum(jnp.where(oh, pz, 0.0), axis=1, keepdims=True)
        ohc = iota_m == i
        ax = jnp.where(ohc, nlx, ax)
        ay = jnp.where(ohc, nly, ay)
        az = jnp.where(ohc, nlz, az)
        return dists, nlx, nly, nlz, ax, ay, az

    st = (dists, lx0, ly0, lz0, ax, ay, az)
    st = lax.fori_loop(1, _M, body, st)
    ax_ref[...] = st[4]
    ay_ref[...] = st[5]
    az_ref[...] = st[6]


def _run_fps(p_soa):
    shp = jax.ShapeDtypeStruct((32, _M), jnp.float32)
    return pl.pallas_call(
        _fps_body,
        out_shape=(shp, shp, shp),
    )(p_soa)


def _g_body(x_ref, v_ref, o_ref):
    o_ref[...] = jnp.dot(x_ref[...], v_ref[...],
                         preferred_element_type=jnp.float32)


def _run_g(x_flat, vt):
    rows = x_flat.shape[0]
    bs = 8192
    return pl.pallas_call(
        _g_body,
        grid=(rows // bs,),
        in_specs=[
            pl.BlockSpec((bs, 3), lambda i: (i, 0)),
            pl.BlockSpec((3, _DIM), lambda i: (0, 0)),
        ],
        out_specs=pl.BlockSpec((bs, _DIM), lambda i: (i, 0)),
        out_shape=jax.ShapeDtypeStruct((rows, _DIM), jnp.float32),
    )(x_flat, vt)


def _sc_kernel_fn():
    mesh = plsc.VectorSubcoreMesh(core_axis_name="c", subcore_axis_name="s")

    @functools.partial(
        pl.kernel,
        mesh=mesh,
        out_type=jax.ShapeDtypeStruct((_GROUPS * _M * _DIM,), jnp.float32),
        scratch_types=[
            pltpu.VMEM((3 * _N,), jnp.float32),
            pltpu.VMEM((3 * _M + 32,), jnp.float32),
            pltpu.VMEM((48,), jnp.int32),
            pltpu.VMEM((64,), jnp.int32),
            pltpu.VMEM((_K,), jnp.int32),
            pltpu.VMEM((_K, _DIM), jnp.float32),
            pltpu.VMEM((_M * _DIM,), jnp.float32),
            pltpu.SemaphoreType.DMA,
        ],
    )
    def sc_kern(pts_hbm, anc_hbm, g_hbm, out_hbm,
                pts_v, anc_v, wrk_v, idx_v, gid_v, rows_v, out_v, sem):
        wid = lax.axis_index("s") * 2 + lax.axis_index("c")
        lane = lax.broadcasted_iota(jnp.int32, (16,), 0)

        for k in range(_GROUPS // _NW):
            gid = wid * (_GROUPS // _NW) + k
            b = gid // 24
            rem = gid - b * 24
            o = rem // 3
            j = rem - o * 3
            frame = jnp.clip(2 * o - 1 + j, 0, 15)
            base = (b * 16 + frame) * _N

            for cc in range(3):
                pltpu.sync_copy(
                    pts_hbm.at[pl.ds(((cc * 4 + b) * 16 + frame) * _N, _N)],
                    pts_v.at[pl.ds(cc * _N, _N)])
                pltpu.sync_copy(
                    anc_hbm.at[pl.ds(((b * 8 + o) * 3 + cc) * _M, _M)],
                    anc_v.at[pl.ds(cc * _M, _M)])

            def per_anchor(a, carry):
                axv = jnp.full((16,), anc_v[pl.ds(a, 16)][0], jnp.float32)
                ayv = jnp.full((16,), anc_v[pl.ds(_M + a, 16)][0],
                               jnp.float32)
                azv = jnp.full((16,), anc_v[pl.ds(2 * _M + a, 16)][0],
                               jnp.float32)

                def chunk_work(ch, cnt):
                    px = pts_v[pl.ds(ch * 16, 16)]
                    py = pts_v[pl.ds(_N + ch * 16, 16)]
                    pz = pts_v[pl.ds(2 * _N + ch * 16, 16)]
                    dx = px - axv
                    dy = py - ayv
                    dz = pz - azv
                    d2 = dx * dx + dy * dy + dz * dz
                    m = d2 < _RR
                    mi = jnp.where(m, 1, 0)
                    wrk_v[pl.ds(16, 16)] = mi
                    s = mi + jnp.where(lane < 12,
                                       wrk_v[pl.ds(20, 16)], 0)
                    wrk_v[pl.ds(16, 16)] = s
                    s = s + jnp.where(lane < 8,
                                      wrk_v[pl.ds(24, 16)], 0)
                    tot = s[0] + s[1] + s[2] + s[3]

                    def app(c):
                        for l in range(16):
                            @pl.when(mi[l] == 1)
                            def _(l=l, c=c):
                                idx_v[pl.ds(c, 16)] = jnp.full(
                                    (16,), ch * 16 + l, jnp.int32)
                            c = c + mi[l]
                        return c

                    return lax.cond(tot > 0, app, lambda c: c, cnt)

                def scan_body(i, cnt):
                    def hot2():
                        c = chunk_work(2 * i, cnt)
                        return chunk_work(2 * i + 1, c)

                    return lax.cond(cnt < _K, hot2, lambda: cnt)

                cnt = lax.fori_loop(0, _N // 32, scan_body, jnp.int32(0))

                v0 = idx_v[pl.ds(0, 16)]
                v1 = idx_v[pl.ds(16, 16)]
                i0 = jnp.where(cnt > 0, v0[0], 0)
                s0 = jnp.where(lane < cnt, v0, i0) + base
                s1 = jnp.where(lane + 16 < cnt, v1, i0) + base
                gid_v[pl.ds(0, 16)] = s0
                gid_v[pl.ds(16, 16)] = s1

                pltpu.async_copy(g_hbm.at[gid_v], rows_v, sem).wait()

                acc = [rows_v[0, pl.ds(c * 16, 16)] for c in range(8)]
                for r in range(1, _K):
                    for c in range(8):
                        acc[c] = jnp.maximum(acc[c],
                                             rows_v[r, pl.ds(c * 16, 16)])
                for c in range(8):
                    out_v[pl.ds(a * _DIM + c * 16, 16)] = acc[c]
                return carry

            lax.fori_loop(0, _M, per_anchor, jnp.int32(0))
            pltpu.sync_copy(out_v, out_hbm.at[pl.ds(gid * _M * _DIM,
                                                    _M * _DIM)])

    return sc_kern


def _epi_body(s0_ref, s1_ref, s2_ref, a2_ref, wdt_ref, wpt_ref, bp_ref,
              o_ref):
    g = pl.program_id(0)
    ts = (g % _TP + 1).astype(jnp.float32)
    wd3 = wdt_ref[3:4, :]
    m = jnp.maximum(jnp.maximum(s0_ref[...] - wd3, s1_ref[...]),
                    s2_ref[...] + wd3)
    a2 = a2_ref[...]
    neg = jnp.dot(a2, wdt_ref[0:3, :], preferred_element_type=jnp.float32)
    pos = jnp.dot(a2, wpt_ref[0:3, :], preferred_element_type=jnp.float32)
    emb = m - neg + pos + ts * wpt_ref[3:4, :] + bp_ref[...]
    o_ref[...] = jnp.maximum(emb, 0.0)


def _run_epilogue(scmax_rows, a2, wdt, wpt, bp):
    def im(j):
        return lambda g: ((g // _TP) * 24 + (g % _TP) * 3 + j, 0)

    return pl.pallas_call(
        _epi_body,
        grid=(32,),
        in_specs=[
            pl.BlockSpec((_M, _DIM), im(0)),
            pl.BlockSpec((_M, _DIM), im(1)),
            pl.BlockSpec((_M, _DIM), im(2)),
            pl.BlockSpec((_M, 3), lambda g: (g, 0)),
            pl.BlockSpec((4, _DIM), lambda g: (0, 0)),
            pl.BlockSpec((4, _DIM), lambda g: (0, 0)),
            pl.BlockSpec((1, _DIM), lambda g: (0, 0)),
        ],
        out_specs=pl.BlockSpec((_M, _DIM), lambda g: (g, 0)),
        out_shape=jax.ShapeDtypeStruct((32 * _M, _DIM), jnp.float32),
    )(scmax_rows, scmax_rows, scmax_rows, a2, wdt, wpt, bp)


@jax.jit
def kernel(x, W_d, W_f, W_pos, b_pos):
    B, T, N, _ = x.shape

    pts_soa = jnp.transpose(x, (3, 0, 1, 2))
    p_fps = pts_soa[:, :, ::2].reshape(3, 32, N)
    x_flat = x.reshape(B * T * N, 3)
    vt = jnp.concatenate([W_d[:, :2], W_d[:, 2:3] + W_f], axis=1).T
    wdt = W_d.T
    wpt = W_pos.T
    bp = b_pos[None, :]

    ax, ay, az = _run_fps(p_fps)
    anc = jnp.stack([ax, ay, az], axis=1).reshape(B, _TP, 3, _M)
    a2 = jnp.stack([ax, ay, az], axis=2).reshape(32 * _M, 3)

    g = _run_g(x_flat, vt)

    pts_1d = pts_soa.reshape(-1)
    anc_1d = anc.reshape(-1)
    scmax = _sc_kernel_fn()(pts_1d, anc_1d, g)
    scmax_rows = scmax.reshape(_GROUPS * _M, _DIM)

    emb = _run_epilogue(scmax_rows, a2, wdt, wpt, bp)
    return emb.reshape(B, _TP * _M, _DIM)


# speedup vs baseline: 1.2629x; 1.0330x over previous
import functools

import jax
import jax.numpy as jnp
from jax import lax
from jax.experimental import pallas as pl
from jax.experimental.pallas import tpu as pltpu
from jax.experimental.pallas import tpu_sc as plsc

_RR = 0.25
_K = 32
_N = 2048
_M = 256
_TP = 8
_DIM = 128
_NW = 32
_GROUPS = 96


def _fps_body(p_ref, ax_ref, ay_ref, az_ref):
    px = p_ref[0]
    py = p_ref[1]
    pz = p_ref[2]
    iota_n = lax.broadcasted_iota(jnp.int32, (32, _N), 1)
    iota_m = lax.broadcasted_iota(jnp.int32, (32, _M), 1)

    lx0 = px[:, 0:1]
    ly0 = py[:, 0:1]
    lz0 = pz[:, 0:1]
    ax = jnp.where(iota_m == 0, lx0, 0.0)
    ay = jnp.where(iota_m == 0, ly0, 0.0)
    az = jnp.where(iota_m == 0, lz0, 0.0)
    dists = jnp.full((32, _N), 1e10, dtype=jnp.float32)

    def body(i, st):
        dists, lx, ly, lz, ax, ay, az = st
        dx = px - lx
        dy = py - ly
        dz = pz - lz
        d = dx * dx + dy * dy + dz * dz
        dists = jnp.minimum(dists, d)
        mx = jnp.max(dists, axis=1, keepdims=True)
        nxt = jnp.min(jnp.where(dists == mx, iota_n, _N), axis=1, keepdims=True)
        oh = iota_n == nxt
        nlx = jnp.sum(jnp.where(oh, px, 0.0), axis=1, keepdims=True)
        nly = jnp.sum(jnp.where(oh, py, 0.0), axis=1, keepdims=True)
        nlz = jnp.sum(jnp.where(oh, pz, 0.0), axis=1, keepdims=True)
        ohc = iota_m == i
        ax = jnp.where(ohc, nlx, ax)
        ay = jnp.where(ohc, nly, ay)
        az = jnp.where(ohc, nlz, az)
        return dists, nlx, nly, nlz, ax, ay, az

    st = (dists, lx0, ly0, lz0, ax, ay, az)
    st = lax.fori_loop(1, _M, body, st)
    ax_ref[...] = st[4]
    ay_ref[...] = st[5]
    az_ref[...] = st[6]


def _run_fps(p_soa):
    shp = jax.ShapeDtypeStruct((32, _M), jnp.float32)
    return pl.pallas_call(
        _fps_body,
        out_shape=(shp, shp, shp),
    )(p_soa)


def _g_body(x_ref, v_ref, o_ref):
    o_ref[...] = jnp.dot(x_ref[...], v_ref[...],
                         preferred_element_type=jnp.float32)


def _run_g(x_flat, vt):
    rows = x_flat.shape[0]
    bs = 8192
    return pl.pallas_call(
        _g_body,
        grid=(rows // bs,),
        in_specs=[
            pl.BlockSpec((bs, 3), lambda i: (i, 0)),
            pl.BlockSpec((3, _DIM), lambda i: (0, 0)),
        ],
        out_specs=pl.BlockSpec((bs, _DIM), lambda i: (i, 0)),
        out_shape=jax.ShapeDtypeStruct((rows, _DIM), jnp.float32),
    )(x_flat, vt)


def _sc_kernel_fn():
    mesh = plsc.VectorSubcoreMesh(core_axis_name="c", subcore_axis_name="s")

    @functools.partial(
        pl.kernel,
        mesh=mesh,
        out_type=jax.ShapeDtypeStruct((_GROUPS * _M * _DIM,), jnp.float32),
        scratch_types=[
            pltpu.VMEM((3 * _N,), jnp.float32),
            pltpu.VMEM((3 * _M + 32,), jnp.float32),
            pltpu.VMEM((48,), jnp.int32),
            pltpu.VMEM((64,), jnp.int32),
            pltpu.VMEM((_K,), jnp.int32),
            pltpu.VMEM((_K, _DIM), jnp.float32),
            pltpu.VMEM((_M * _DIM,), jnp.float32),
            pltpu.SemaphoreType.DMA,
        ],
    )
    def sc_kern(pts_hbm, anc_hbm, g_hbm, out_hbm,
                pts_v, anc_v, wrk_v, idx_v, gid_v, rows_v, out_v, sem):
        wid = lax.axis_index("s") * 2 + lax.axis_index("c")
        lane = lax.broadcasted_iota(jnp.int32, (16,), 0)

        for k in range(_GROUPS // _NW):
            gid = wid * (_GROUPS // _NW) + k
            b = gid // 24
            rem = gid - b * 24
            o = rem // 3
            j = rem - o * 3
            frame = jnp.clip(2 * o - 1 + j, 0, 15)
            base = (b * 16 + frame) * _N

            for cc in range(3):
                pltpu.sync_copy(
                    pts_hbm.at[pl.ds(((cc * 4 + b) * 16 + frame) * _N, _N)],
                    pts_v.at[pl.ds(cc * _N, _N)])
                pltpu.sync_copy(
                    anc_hbm.at[pl.ds(((b * 8 + o) * 3 + cc) * _M, _M)],
                    anc_v.at[pl.ds(cc * _M, _M)])

            def per_anchor(a, carry):
                axv = jnp.full((16,), anc_v[pl.ds(a, 16)][0], jnp.float32)
                ayv = jnp.full((16,), anc_v[pl.ds(_M + a, 16)][0],
                               jnp.float32)
                azv = jnp.full((16,), anc_v[pl.ds(2 * _M + a, 16)][0],
                               jnp.float32)

                def chunk_work(ch, cnt):
                    px = pts_v[pl.ds(ch * 16, 16)]
                    py = pts_v[pl.ds(_N + ch * 16, 16)]
                    pz = pts_v[pl.ds(2 * _N + ch * 16, 16)]
                    dx = px - axv
                    dy = py - ayv
                    dz = pz - azv
                    d2 = dx * dx + dy * dy + dz * dz
                    m = d2 < _RR
                    mi = jnp.where(m, 1, 0)
                    wrk_v[pl.ds(16, 16)] = mi
                    s = mi + jnp.where(lane < 12,
                                       wrk_v[pl.ds(20, 16)], 0)
                    wrk_v[pl.ds(16, 16)] = s
                    s = s + jnp.where(lane < 8,
                                      wrk_v[pl.ds(24, 16)], 0)
                    tot = s[0] + s[1] + s[2] + s[3]

                    def app(c):
                        for l in range(16):
                            @pl.when(mi[l] == 1)
                            def _(l=l, c=c):
                                idx_v[pl.ds(c, 16)] = jnp.full(
                                    (16,), ch * 16 + l, jnp.int32)
                            c = c + mi[l]
                        return c

                    return lax.cond(tot > 0, app, lambda c: c, cnt)

                def scan_body(i, cnt):
                    def hot2():
                        c = chunk_work(4 * i, cnt)
                        c = chunk_work(4 * i + 1, c)
                        c = chunk_work(4 * i + 2, c)
                        return chunk_work(4 * i + 3, c)

                    return lax.cond(cnt < _K, hot2, lambda: cnt)

                cnt = lax.fori_loop(0, _N // 64, scan_body, jnp.int32(0))

                v0 = idx_v[pl.ds(0, 16)]
                v1 = idx_v[pl.ds(16, 16)]
                i0 = jnp.where(cnt > 0, v0[0], 0)
                s0 = jnp.where(lane < cnt, v0, i0) + base
                s1 = jnp.where(lane + 16 < cnt, v1, i0) + base
                gid_v[pl.ds(0, 16)] = s0
                gid_v[pl.ds(16, 16)] = s1

                pltpu.async_copy(g_hbm.at[gid_v], rows_v, sem).wait()

                acc = [rows_v[0, pl.ds(c * 16, 16)] for c in range(8)]
                for r in range(1, _K):
                    for c in range(8):
                        acc[c] = jnp.maximum(acc[c],
                                             rows_v[r, pl.ds(c * 16, 16)])
                for c in range(8):
                    out_v[pl.ds(a * _DIM + c * 16, 16)] = acc[c]
                return carry

            lax.fori_loop(0, _M, per_anchor, jnp.int32(0))
            pltpu.sync_copy(out_v, out_hbm.at[pl.ds(gid * _M * _DIM,
                                                    _M * _DIM)])

    return sc_kern


def _epi_body(s0_ref, s1_ref, s2_ref, a2_ref, wdt_ref, wpt_ref, bp_ref,
              o_ref):
    g = pl.program_id(0)
    ts = (g % _TP + 1).astype(jnp.float32)
    wd3 = wdt_ref[3:4, :]
    m = jnp.maximum(jnp.maximum(s0_ref[...] - wd3, s1_ref[...]),
                    s2_ref[...] + wd3)
    a2 = a2_ref[...]
    neg = jnp.dot(a2, wdt_ref[0:3, :], preferred_element_type=jnp.float32)
    pos = jnp.dot(a2, wpt_ref[0:3, :], preferred_element_type=jnp.float32)
    emb = m - neg + pos + ts * wpt_ref[3:4, :] + bp_ref[...]
    o_ref[...] = jnp.maximum(emb, 0.0)


def _run_epilogue(scmax_rows, a2, wdt, wpt, bp):
    def im(j):
        return lambda g: ((g // _TP) * 24 + (g % _TP) * 3 + j, 0)

    return pl.pallas_call(
        _epi_body,
        grid=(32,),
        in_specs=[
            pl.BlockSpec((_M, _DIM), im(0)),
            pl.BlockSpec((_M, _DIM), im(1)),
            pl.BlockSpec((_M, _DIM), im(2)),
            pl.BlockSpec((_M, 3), lambda g: (g, 0)),
            pl.BlockSpec((4, _DIM), lambda g: (0, 0)),
            pl.BlockSpec((4, _DIM), lambda g: (0, 0)),
            pl.BlockSpec((1, _DIM), lambda g: (0, 0)),
        ],
        out_specs=pl.BlockSpec((_M, _DIM), lambda g: (g, 0)),
        out_shape=jax.ShapeDtypeStruct((32 * _M, _DIM), jnp.float32),
    )(scmax_rows, scmax_rows, scmax_rows, a2, wdt, wpt, bp)


@jax.jit
def kernel(x, W_d, W_f, W_pos, b_pos):
    B, T, N, _ = x.shape

    pts_soa = jnp.transpose(x, (3, 0, 1, 2))
    p_fps = pts_soa[:, :, ::2].reshape(3, 32, N)
    x_flat = x.reshape(B * T * N, 3)
    vt = jnp.concatenate([W_d[:, :2], W_d[:, 2:3] + W_f], axis=1).T
    wdt = W_d.T
    wpt = W_pos.T
    bp = b_pos[None, :]

    ax, ay, az = _run_fps(p_fps)
    anc = jnp.stack([ax, ay, az], axis=1).reshape(B, _TP, 3, _M)
    a2 = jnp.stack([ax, ay, az], axis=2).reshape(32 * _M, 3)

    g = _run_g(x_flat, vt)

    pts_1d = pts_soa.reshape(-1)
    anc_1d = anc.reshape(-1)
    scmax = _sc_kernel_fn()(pts_1d, anc_1d, g)
    scmax_rows = scmax.reshape(_GROUPS * _M, _DIM)

    emb = _run_epilogue(scmax_rows, a2, wdt, wpt, bp)
    return emb.reshape(B, _TP * _M, _DIM)


# speedup vs baseline: 1.5870x; 1.2566x over previous
import functools

import jax
import jax.numpy as jnp
from jax import lax
from jax.experimental import pallas as pl
from jax.experimental.pallas import tpu as pltpu
from jax.experimental.pallas import tpu_sc as plsc

_RR = 0.25
_K = 32
_N = 2048
_M = 256
_TP = 8
_DIM = 128
_NW = 32
_GROUPS = 96


def _fps_body(p_ref, ax_ref, ay_ref, az_ref):
    px = p_ref[0]
    py = p_ref[1]
    pz = p_ref[2]
    iota_n = lax.broadcasted_iota(jnp.int32, (32, _N), 1)
    iota_m = lax.broadcasted_iota(jnp.int32, (32, _M), 1)

    lx0 = px[:, 0:1]
    ly0 = py[:, 0:1]
    lz0 = pz[:, 0:1]
    ax = jnp.where(iota_m == 0, lx0, 0.0)
    ay = jnp.where(iota_m == 0, ly0, 0.0)
    az = jnp.where(iota_m == 0, lz0, 0.0)
    dists = jnp.full((32, _N), 1e10, dtype=jnp.float32)

    def body(i, st):
        dists, lx, ly, lz, ax, ay, az = st
        dx = px - lx
        dy = py - ly
        dz = pz - lz
        d = dx * dx + dy * dy + dz * dz
        dists = jnp.minimum(dists, d)
        mx = jnp.max(dists, axis=1, keepdims=True)
        nxt = jnp.min(jnp.where(dists == mx, iota_n, _N), axis=1, keepdims=True)
        oh = iota_n == nxt
        nlx = jnp.sum(jnp.where(oh, px, 0.0), axis=1, keepdims=True)
        nly = jnp.sum(jnp.where(oh, py, 0.0), axis=1, keepdims=True)
        nlz = jnp.sum(jnp.where(oh, pz, 0.0), axis=1, keepdims=True)
        ohc = iota_m == i
        ax = jnp.where(ohc, nlx, ax)
        ay = jnp.where(ohc, nly, ay)
        az = jnp.where(ohc, nlz, az)
        return dists, nlx, nly, nlz, ax, ay, az

    st = (dists, lx0, ly0, lz0, ax, ay, az)
    st = lax.fori_loop(1, _M, body, st)
    ax_ref[...] = st[4]
    ay_ref[...] = st[5]
    az_ref[...] = st[6]


def _run_fps(p_soa):
    shp = jax.ShapeDtypeStruct((32, _M), jnp.float32)
    return pl.pallas_call(
        _fps_body,
        out_shape=(shp, shp, shp),
    )(p_soa)


def _g_body(x_ref, v_ref, o_ref):
    o_ref[...] = jnp.dot(x_ref[...], v_ref[...],
                         preferred_element_type=jnp.float32)


def _run_g(x_flat, vt):
    rows = x_flat.shape[0]
    bs = 8192
    return pl.pallas_call(
        _g_body,
        grid=(rows // bs,),
        in_specs=[
            pl.BlockSpec((bs, 3), lambda i: (i, 0)),
            pl.BlockSpec((3, _DIM), lambda i: (0, 0)),
        ],
        out_specs=pl.BlockSpec((bs, _DIM), lambda i: (i, 0)),
        out_shape=jax.ShapeDtypeStruct((rows, _DIM), jnp.float32),
    )(x_flat, vt)


def _sc_kernel_fn():
    mesh = plsc.VectorSubcoreMesh(core_axis_name="c", subcore_axis_name="s")

    @functools.partial(
        pl.kernel,
        mesh=mesh,
        out_type=jax.ShapeDtypeStruct((_GROUPS * _M * _DIM,), jnp.float32),
        scratch_types=[
            pltpu.VMEM((3 * _N,), jnp.float32),
            pltpu.VMEM((3 * _M + 32,), jnp.float32),
            pltpu.VMEM((48,), jnp.int32),
            pltpu.VMEM((64,), jnp.int32),
            pltpu.VMEM((_M * _K,), jnp.int32),
            pltpu.VMEM((_K,), jnp.int32),
            pltpu.VMEM((_K,), jnp.int32),
            pltpu.VMEM((_K,), jnp.int32),
            pltpu.VMEM((_K,), jnp.int32),
            pltpu.VMEM((_K, _DIM), jnp.float32),
            pltpu.VMEM((_K, _DIM), jnp.float32),
            pltpu.VMEM((_K, _DIM), jnp.float32),
            pltpu.VMEM((_K, _DIM), jnp.float32),
            pltpu.VMEM((_M * _DIM,), jnp.float32),
            pltpu.SemaphoreType.DMA,
            pltpu.SemaphoreType.DMA,
            pltpu.SemaphoreType.DMA,
            pltpu.SemaphoreType.DMA,
        ],
    )
    def sc_kern(pts_hbm, anc_hbm, g_hbm, out_hbm,
                pts_v, anc_v, wrk_v, idx_v, gidall_v,
                g0, g1, g2, g3, r0, r1, r2, r3, out_v,
                sm0, sm1, sm2, sm3):
        wid = lax.axis_index("s") * 2 + lax.axis_index("c")
        lane = lax.broadcasted_iota(jnp.int32, (16,), 0)
        gbufs = [g0, g1, g2, g3]
        rbufs = [r0, r1, r2, r3]
        sems = [sm0, sm1, sm2, sm3]

        def group_body(k, carry):
            gid = wid * (_GROUPS // _NW) + k
            b = gid // 24
            rem = gid - b * 24
            o = rem // 3
            j = rem - o * 3
            frame = jnp.clip(2 * o - 1 + j, 0, 15)
            base = (b * 16 + frame) * _N

            for cc in range(3):
                pltpu.sync_copy(
                    pts_hbm.at[pl.ds(((cc * 4 + b) * 16 + frame) * _N, _N)],
                    pts_v.at[pl.ds(cc * _N, _N)])
                pltpu.sync_copy(
                    anc_hbm.at[pl.ds(((b * 8 + o) * 3 + cc) * _M, _M)],
                    anc_v.at[pl.ds(cc * _M, _M)])

            def per_anchor(a, carry):
                axv = jnp.full((16,), anc_v[pl.ds(a, 16)][0], jnp.float32)
                ayv = jnp.full((16,), anc_v[pl.ds(_M + a, 16)][0],
                               jnp.float32)
                azv = jnp.full((16,), anc_v[pl.ds(2 * _M + a, 16)][0],
                               jnp.float32)

                def chunk_work(ch, cnt):
                    px = pts_v[pl.ds(ch * 16, 16)]
                    py = pts_v[pl.ds(_N + ch * 16, 16)]
                    pz = pts_v[pl.ds(2 * _N + ch * 16, 16)]
                    dx = px - axv
                    dy = py - ayv
                    dz = pz - azv
                    d2 = dx * dx + dy * dy + dz * dz
                    m = d2 < _RR
                    mi = jnp.where(m, 1, 0)
                    wrk_v[pl.ds(16, 16)] = mi
                    s = mi + jnp.where(lane < 12,
                                       wrk_v[pl.ds(20, 16)], 0)
                    wrk_v[pl.ds(16, 16)] = s
                    s = s + jnp.where(lane < 8,
                                      wrk_v[pl.ds(24, 16)], 0)
                    tot = s[0] + s[1] + s[2] + s[3]

                    def app(c):
                        for l in range(16):
                            @pl.when(mi[l] == 1)
                            def _(l=l, c=c):
                                idx_v[pl.ds(c, 16)] = jnp.full(
                                    (16,), ch * 16 + l, jnp.int32)
                            c = c + mi[l]
                        return c

                    return lax.cond(tot > 0, app, lambda c: c, cnt)

                def scan_body(i, cnt):
                    def hot2():
                        c = chunk_work(4 * i, cnt)
                        c = chunk_work(4 * i + 1, c)
                        c = chunk_work(4 * i + 2, c)
                        return chunk_work(4 * i + 3, c)

                    return lax.cond(cnt < _K, hot2, lambda: cnt)

                cnt = lax.fori_loop(0, _N // 64, scan_body, jnp.int32(0))

                v0 = idx_v[pl.ds(0, 16)]
                v1 = idx_v[pl.ds(16, 16)]
                i0 = jnp.where(cnt > 0, v0[0], 0)
                s0 = jnp.where(lane < cnt, v0, i0) + base
                s1 = jnp.where(lane + 16 < cnt, v1, i0) + base
                gidall_v[pl.ds(a * _K, 16)] = s0
                gidall_v[pl.ds(a * _K + 16, 16)] = s1
                return carry

            lax.fori_loop(0, _M, per_anchor, jnp.int32(0))

            def fill(b, a):
                gbufs[b][pl.ds(0, 16)] = gidall_v[pl.ds(a * _K, 16)]
                gbufs[b][pl.ds(16, 16)] = gidall_v[pl.ds(a * _K + 16, 16)]
                pltpu.async_copy(g_hbm.at[gbufs[b]], rbufs[b], sems[b])

            for b in range(4):
                fill(b, b)

            def gm_body(i, carry2):
                for b in range(4):
                    a = i * 4 + b
                    pltpu.make_async_copy(g_hbm.at[gbufs[b]], rbufs[b],
                                          sems[b]).wait()
                    rv = rbufs[b]
                    acc = [rv[0, pl.ds(c * 16, 16)] for c in range(8)]
                    for r in range(1, _K):
                        for c in range(8):
                            acc[c] = jnp.maximum(acc[c],
                                                 rv[r, pl.ds(c * 16, 16)])
                    for c in range(8):
                        out_v[pl.ds(a * _DIM + c * 16, 16)] = acc[c]

                    @pl.when(a + 4 < _M)
                    def _(b=b, a=a):
                        fill(b, a + 4)
                return carry2

            lax.fori_loop(0, _M // 4, gm_body, jnp.int32(0))
            pltpu.sync_copy(out_v, out_hbm.at[pl.ds(gid * _M * _DIM,
                                                    _M * _DIM)])
            return carry

        lax.fori_loop(0, _GROUPS // _NW, group_body, jnp.int32(0))

    return sc_kern


def _epi_body(s0_ref, s1_ref, s2_ref, a2_ref, wdt_ref, wpt_ref, bp_ref,
              o_ref):
    g = pl.program_id(0)
    ts = (g % _TP + 1).astype(jnp.float32)
    wd3 = wdt_ref[3:4, :]
    m = jnp.maximum(jnp.maximum(s0_ref[...] - wd3, s1_ref[...]),
                    s2_ref[...] + wd3)
    a2 = a2_ref[...]
    neg = jnp.dot(a2, wdt_ref[0:3, :], preferred_element_type=jnp.float32)
    pos = jnp.dot(a2, wpt_ref[0:3, :], preferred_element_type=jnp.float32)
    emb = m - neg + pos + ts * wpt_ref[3:4, :] + bp_ref[...]
    o_ref[...] = jnp.maximum(emb, 0.0)


def _run_epilogue(scmax_rows, a2, wdt, wpt, bp):
    def im(j):
        return lambda g: ((g // _TP) * 24 + (g % _TP) * 3 + j, 0)

    return pl.pallas_call(
        _epi_body,
        grid=(32,),
        in_specs=[
            pl.BlockSpec((_M, _DIM), im(0)),
            pl.BlockSpec((_M, _DIM), im(1)),
            pl.BlockSpec((_M, _DIM), im(2)),
            pl.BlockSpec((_M, 3), lambda g: (g, 0)),
            pl.BlockSpec((4, _DIM), lambda g: (0, 0)),
            pl.BlockSpec((4, _DIM), lambda g: (0, 0)),
            pl.BlockSpec((1, _DIM), lambda g: (0, 0)),
        ],
        out_specs=pl.BlockSpec((_M, _DIM), lambda g: (g, 0)),
        out_shape=jax.ShapeDtypeStruct((32 * _M, _DIM), jnp.float32),
    )(scmax_rows, scmax_rows, scmax_rows, a2, wdt, wpt, bp)


@jax.jit
def kernel(x, W_d, W_f, W_pos, b_pos):
    B, T, N, _ = x.shape

    pts_soa = jnp.transpose(x, (3, 0, 1, 2))
    p_fps = pts_soa[:, :, ::2].reshape(3, 32, N)
    x_flat = x.reshape(B * T * N, 3)
    vt = jnp.concatenate([W_d[:, :2], W_d[:, 2:3] + W_f], axis=1).T
    wdt = W_d.T
    wpt = W_pos.T
    bp = b_pos[None, :]

    ax, ay, az = _run_fps(p_fps)
    anc = jnp.stack([ax, ay, az], axis=1).reshape(B, _TP, 3, _M)
    a2 = jnp.stack([ax, ay, az], axis=2).reshape(32 * _M, 3)

    g = _run_g(x_flat, vt)

    pts_1d = pts_soa.reshape(-1)
    anc_1d = anc.reshape(-1)
    scmax = _sc_kernel_fn()(pts_1d, anc_1d, g)
    scmax_rows = scmax.reshape(_GROUPS * _M, _DIM)

    emb = _run_epilogue(scmax_rows, a2, wdt, wpt, bp)
    return emb.reshape(B, _TP * _M, _DIM)


# speedup vs baseline: 1.6239x; 1.0233x over previous
import functools

import jax
import jax.numpy as jnp
from jax import lax
from jax.experimental import pallas as pl
from jax.experimental.pallas import tpu as pltpu
from jax.experimental.pallas import tpu_sc as plsc

_RR = 0.25
_K = 32
_N = 2048
_M = 256
_TP = 8
_DIM = 128
_NW = 32
_GROUPS = 96


def _fps_body(p_ref, ax_ref, ay_ref, az_ref):
    px = p_ref[0]
    py = p_ref[1]
    pz = p_ref[2]
    iota_n = lax.broadcasted_iota(jnp.int32, (32, _N), 1)
    iota_m = lax.broadcasted_iota(jnp.int32, (32, _M), 1)

    lx0 = px[:, 0:1]
    ly0 = py[:, 0:1]
    lz0 = pz[:, 0:1]
    ax = jnp.where(iota_m == 0, lx0, 0.0)
    ay = jnp.where(iota_m == 0, ly0, 0.0)
    az = jnp.where(iota_m == 0, lz0, 0.0)
    dists = jnp.full((32, _N), 1e10, dtype=jnp.float32)

    def body(i, st):
        dists, lx, ly, lz, ax, ay, az = st
        dx = px - lx
        dy = py - ly
        dz = pz - lz
        d = dx * dx + dy * dy + dz * dz
        dists = jnp.minimum(dists, d)
        mx = jnp.max(dists, axis=1, keepdims=True)
        nxt = jnp.min(jnp.where(dists == mx, iota_n, _N), axis=1, keepdims=True)
        oh = iota_n == nxt
        nlx = jnp.sum(jnp.where(oh, px, 0.0), axis=1, keepdims=True)
        nly = jnp.sum(jnp.where(oh, py, 0.0), axis=1, keepdims=True)
        nlz = jnp.sum(jnp.where(oh, pz, 0.0), axis=1, keepdims=True)
        ohc = iota_m == i
        ax = jnp.where(ohc, nlx, ax)
        ay = jnp.where(ohc, nly, ay)
        az = jnp.where(ohc, nlz, az)
        return dists, nlx, nly, nlz, ax, ay, az

    st = (dists, lx0, ly0, lz0, ax, ay, az)
    st = lax.fori_loop(1, _M, body, st)
    ax_ref[...] = st[4]
    ay_ref[...] = st[5]
    az_ref[...] = st[6]


def _run_fps(p_soa):
    shp = jax.ShapeDtypeStruct((32, _M), jnp.float32)
    return pl.pallas_call(
        _fps_body,
        out_shape=(shp, shp, shp),
    )(p_soa)


def _g_body(x_ref, v_ref, o_ref):
    o_ref[...] = jnp.dot(x_ref[...], v_ref[...],
                         preferred_element_type=jnp.float32)


def _run_g(x_flat, vt):
    rows = x_flat.shape[0]
    bs = 8192
    return pl.pallas_call(
        _g_body,
        grid=(rows // bs,),
        in_specs=[
            pl.BlockSpec((bs, 3), lambda i: (i, 0)),
            pl.BlockSpec((3, _DIM), lambda i: (0, 0)),
        ],
        out_specs=pl.BlockSpec((bs, _DIM), lambda i: (i, 0)),
        out_shape=jax.ShapeDtypeStruct((rows, _DIM), jnp.float32),
    )(x_flat, vt)


def _sc_kernel_fn():
    mesh = plsc.VectorSubcoreMesh(core_axis_name="c", subcore_axis_name="s")

    @functools.partial(
        pl.kernel,
        mesh=mesh,
        out_type=jax.ShapeDtypeStruct((_GROUPS * _M * _DIM,), jnp.float32),
        scratch_types=[
            pltpu.VMEM((3 * _N,), jnp.float32),
            pltpu.VMEM((3 * _M + 32,), jnp.float32),
            pltpu.VMEM((48,), jnp.int32),
            pltpu.VMEM((64,), jnp.int32),
            pltpu.VMEM((_M * _K,), jnp.int32),
            pltpu.VMEM((_K,), jnp.int32),
            pltpu.VMEM((_K,), jnp.int32),
            pltpu.VMEM((_K,), jnp.int32),
            pltpu.VMEM((_K,), jnp.int32),
            pltpu.VMEM((_K, _DIM), jnp.float32),
            pltpu.VMEM((_K, _DIM), jnp.float32),
            pltpu.VMEM((_K, _DIM), jnp.float32),
            pltpu.VMEM((_K, _DIM), jnp.float32),
            pltpu.VMEM((_M * _DIM,), jnp.float32),
            pltpu.SemaphoreType.DMA,
            pltpu.SemaphoreType.DMA,
            pltpu.SemaphoreType.DMA,
            pltpu.SemaphoreType.DMA,
        ],
    )
    def sc_kern(pts_hbm, anc_hbm, g_hbm, out_hbm,
                pts_v, anc_v, wrk_v, idx_v, gidall_v,
                g0, g1, g2, g3, r0, r1, r2, r3, out_v,
                sm0, sm1, sm2, sm3):
        wid = lax.axis_index("s") * 2 + lax.axis_index("c")
        lane = lax.broadcasted_iota(jnp.int32, (16,), 0)
        gbufs = [g0, g1, g2, g3]
        rbufs = [r0, r1, r2, r3]
        sems = [sm0, sm1, sm2, sm3]

        def group_body(k, carry):
            gid = wid * (_GROUPS // _NW) + k
            b = gid // 24
            rem = gid - b * 24
            o = rem // 3
            j = rem - o * 3
            frame = jnp.clip(2 * o - 1 + j, 0, 15)
            base = (b * 16 + frame) * _N

            for cc in range(3):
                pltpu.sync_copy(
                    pts_hbm.at[pl.ds(((cc * 4 + b) * 16 + frame) * _N, _N)],
                    pts_v.at[pl.ds(cc * _N, _N)])
                pltpu.sync_copy(
                    anc_hbm.at[pl.ds(((b * 8 + o) * 3 + cc) * _M, _M)],
                    anc_v.at[pl.ds(cc * _M, _M)])

            def per_anchor(a, carry):
                axv = jnp.full((16,), anc_v[pl.ds(a, 16)][0], jnp.float32)
                ayv = jnp.full((16,), anc_v[pl.ds(_M + a, 16)][0],
                               jnp.float32)
                azv = jnp.full((16,), anc_v[pl.ds(2 * _M + a, 16)][0],
                               jnp.float32)

                def chunk_work(ch, cnt):
                    px = pts_v[pl.ds(ch * 16, 16)]
                    py = pts_v[pl.ds(_N + ch * 16, 16)]
                    pz = pts_v[pl.ds(2 * _N + ch * 16, 16)]
                    dx = px - axv
                    dy = py - ayv
                    dz = pz - azv
                    d2 = dx * dx + dy * dy + dz * dz
                    m = d2 < _RR
                    mi = jnp.where(m, 1, 0)
                    wrk_v[pl.ds(16, 16)] = mi
                    s = mi + jnp.where(lane < 12,
                                       wrk_v[pl.ds(20, 16)], 0)
                    wrk_v[pl.ds(16, 16)] = s
                    s = s + jnp.where(lane < 8,
                                      wrk_v[pl.ds(24, 16)], 0)
                    tot = s[0] + s[1] + s[2] + s[3]

                    def app(c):
                        for l in range(16):
                            @pl.when(mi[l] == 1)
                            def _(l=l, c=c):
                                idx_v[pl.ds(c, 16)] = jnp.full(
                                    (16,), ch * 16 + l, jnp.int32)
                            c = c + mi[l]
                        return c

                    return lax.cond(tot > 0, app, lambda c: c, cnt)

                def scan_body(i, cnt):
                    def hot2():
                        c = cnt
                        for u in range(8):
                            c = chunk_work(8 * i + u, c)
                        return c

                    return lax.cond(cnt < _K, hot2, lambda: cnt)

                cnt = lax.fori_loop(0, _N // 128, scan_body, jnp.int32(0))

                v0 = idx_v[pl.ds(0, 16)]
                v1 = idx_v[pl.ds(16, 16)]
                i0 = jnp.where(cnt > 0, v0[0], 0)
                s0 = jnp.where(lane < cnt, v0, i0) + base
                s1 = jnp.where(lane + 16 < cnt, v1, i0) + base
                gidall_v[pl.ds(a * _K, 16)] = s0
                gidall_v[pl.ds(a * _K + 16, 16)] = s1
                return carry

            lax.fori_loop(0, _M, per_anchor, jnp.int32(0))

            def fill(b, a):
                gbufs[b][pl.ds(0, 16)] = gidall_v[pl.ds(a * _K, 16)]
                gbufs[b][pl.ds(16, 16)] = gidall_v[pl.ds(a * _K + 16, 16)]
                pltpu.async_copy(g_hbm.at[gbufs[b]], rbufs[b], sems[b])

            for b in range(4):
                fill(b, b)

            def gm_body(i, carry2):
                for b in range(4):
                    a = i * 4 + b
                    pltpu.make_async_copy(g_hbm.at[gbufs[b]], rbufs[b],
                                          sems[b]).wait()
                    rv = rbufs[b]
                    acc = [rv[0, pl.ds(c * 16, 16)] for c in range(8)]
                    for r in range(1, _K):
                        for c in range(8):
                            acc[c] = jnp.maximum(acc[c],
                                                 rv[r, pl.ds(c * 16, 16)])
                    for c in range(8):
                        out_v[pl.ds(a * _DIM + c * 16, 16)] = acc[c]

                    @pl.when(a + 4 < _M)
                    def _(b=b, a=a):
                        fill(b, a + 4)
                return carry2

            lax.fori_loop(0, _M // 4, gm_body, jnp.int32(0))
            pltpu.sync_copy(out_v, out_hbm.at[pl.ds(gid * _M * _DIM,
                                                    _M * _DIM)])
            return carry

        lax.fori_loop(0, _GROUPS // _NW, group_body, jnp.int32(0))

    return sc_kern


def _epi_body(s0_ref, s1_ref, s2_ref, a2_ref, wdt_ref, wpt_ref, bp_ref,
              o_ref):
    g = pl.program_id(0)
    ts = (g % _TP + 1).astype(jnp.float32)
    wd3 = wdt_ref[3:4, :]
    m = jnp.maximum(jnp.maximum(s0_ref[...] - wd3, s1_ref[...]),
                    s2_ref[...] + wd3)
    a2 = a2_ref[...]
    neg = jnp.dot(a2, wdt_ref[0:3, :], preferred_element_type=jnp.float32)
    pos = jnp.dot(a2, wpt_ref[0:3, :], preferred_element_type=jnp.float32)
    emb = m - neg + pos + ts * wpt_ref[3:4, :] + bp_ref[...]
    o_ref[...] = jnp.maximum(emb, 0.0)


def _run_epilogue(scmax_rows, a2, wdt, wpt, bp):
    def im(j):
        return lambda g: ((g // _TP) * 24 + (g % _TP) * 3 + j, 0)

    return pl.pallas_call(
        _epi_body,
        grid=(32,),
        in_specs=[
            pl.BlockSpec((_M, _DIM), im(0)),
            pl.BlockSpec((_M, _DIM), im(1)),
            pl.BlockSpec((_M, _DIM), im(2)),
            pl.BlockSpec((_M, 3), lambda g: (g, 0)),
            pl.BlockSpec((4, _DIM), lambda g: (0, 0)),
            pl.BlockSpec((4, _DIM), lambda g: (0, 0)),
            pl.BlockSpec((1, _DIM), lambda g: (0, 0)),
        ],
        out_specs=pl.BlockSpec((_M, _DIM), lambda g: (g, 0)),
        out_shape=jax.ShapeDtypeStruct((32 * _M, _DIM), jnp.float32),
    )(scmax_rows, scmax_rows, scmax_rows, a2, wdt, wpt, bp)


@jax.jit
def kernel(x, W_d, W_f, W_pos, b_pos):
    B, T, N, _ = x.shape

    pts_soa = jnp.transpose(x, (3, 0, 1, 2))
    p_fps = pts_soa[:, :, ::2].reshape(3, 32, N)
    x_flat = x.reshape(B * T * N, 3)
    vt = jnp.concatenate([W_d[:, :2], W_d[:, 2:3] + W_f], axis=1).T
    wdt = W_d.T
    wpt = W_pos.T
    bp = b_pos[None, :]

    ax, ay, az = _run_fps(p_fps)
    anc = jnp.stack([ax, ay, az], axis=1).reshape(B, _TP, 3, _M)
    a2 = jnp.stack([ax, ay, az], axis=2).reshape(32 * _M, 3)

    g = _run_g(x_flat, vt)

    pts_1d = pts_soa.reshape(-1)
    anc_1d = anc.reshape(-1)
    scmax = _sc_kernel_fn()(pts_1d, anc_1d, g)
    scmax_rows = scmax.reshape(_GROUPS * _M, _DIM)

    emb = _run_epilogue(scmax_rows, a2, wdt, wpt, bp)
    return emb.reshape(B, _TP * _M, _DIM)
